# Initial kernel scaffold; baseline (speedup 1.0000x reference)
import jax
import jax.numpy as jnp
from jax.experimental import pallas as pl


def kernel(loc_data, conf_data, landm_data, gender_data, priors, targets):
    raise NotImplementedError("write your pallas kernel here")


# speedup vs baseline: 104.6314x; 104.6314x over previous
import functools

import jax
import jax.numpy as jnp
from jax.experimental import pallas as pl

_NUM_CLASSES = 2
_THRESHOLD = 0.35
_NEGPOS_RATIO = 7.0
_V0 = 0.1
_V1 = 0.2


def _smooth_l1(x, t):
    d = x - t
    ad = jnp.abs(d)
    return jnp.where(ad < 1.0, 0.5 * d * d, ad - 0.5)


def _match_kernel(loc_ref, conf_ref, landm_ref, gender_ref, pri_ref, tgt_ref,
                  tgtT_ref, stats_ref, mlc_ref):
    T, P = tgt_ref.shape[1], pri_ref.shape[1]
    pri = pri_ref[...]
    tgtb = tgt_ref[0]
    tgtT = tgtT_ref[0]

    pcx, pcy = pri[0:1, :], pri[1:2, :]
    pw, ph = pri[2:3, :], pri[3:4, :]
    xb1 = pcx - pw / 2
    yb1 = pcy - ph / 2
    xb2 = pcx + pw / 2
    yb2 = pcy + ph / 2

    tx1, ty1 = tgtb[:, 0:1], tgtb[:, 1:2]
    tx2, ty2 = tgtb[:, 2:3], tgtb[:, 3:4]

    ix1 = jnp.maximum(tx1, xb1)
    iy1 = jnp.maximum(ty1, yb1)
    ix2 = jnp.minimum(tx2, xb2)
    iy2 = jnp.minimum(ty2, yb2)
    inter = jnp.maximum(ix2 - ix1, 0.0) * jnp.maximum(iy2 - iy1, 0.0)
    area_t = (tx2 - tx1) * (ty2 - ty1)
    area_p = (xb2 - xb1) * (yb2 - yb1)
    ov = inter / (area_t + area_p - inter)

    lane_i = jax.lax.broadcasted_iota(jnp.int32, (T, P), 1)
    sub_i = jax.lax.broadcasted_iota(jnp.int32, (T, P), 0)

    bpo = jnp.max(ov, axis=1, keepdims=True)
    bpi = jnp.min(jnp.where(ov == bpo, lane_i, P), axis=1, keepdims=True)
    valid = (bpo >= 0.2).astype(jnp.int32)

    bto = jnp.max(ov, axis=0, keepdims=True)
    bti = jnp.min(jnp.where(ov == bto, sub_i, T), axis=0, keepdims=True)

    eq = bpi == lane_i
    code = jnp.where(eq, 2 * sub_i + valid, -1)
    last_code = jnp.max(code, axis=0, keepdims=True)
    any_eq = last_code >= 0
    last_j = last_code // 2
    valid_last = (last_code - 2 * last_j) == 1
    bto2 = jnp.where(any_eq & valid_last, 2.0, bto)
    bti2 = jnp.where(any_eq, last_j, bti)

    onehot = (bti2 == sub_i).astype(jnp.float32)
    matched = jax.lax.dot_general(
        tgtT, onehot, (((1,), (0,)), ((), ())),
        precision=jax.lax.Precision.HIGHEST,
        preferred_element_type=jnp.float32)

    label_g = matched[14:15]
    gender_g = matched[15:16]
    conf = jnp.where(bto2 < _THRESHOLD, 0.0, label_g)
    pos = conf != 0.0
    pos1 = conf > 0.0

    mx1, my1, mx2, my2 = matched[0:1], matched[1:2], matched[2:3], matched[3:4]
    g_cx = ((mx1 + mx2) / 2 - pcx) / (_V0 * pw)
    g_cy = ((my1 + my2) / 2 - pcy) / (_V0 * ph)
    g_w = jnp.log((mx2 - mx1) / pw) / _V1
    g_h = jnp.log((my2 - my1) / ph) / _V1
    loc = loc_ref[0]
    ll = (jnp.where(pos, _smooth_l1(loc[0:1], g_cx), 0.0)
          + jnp.where(pos, _smooth_l1(loc[1:2], g_cy), 0.0)
          + jnp.where(pos, _smooth_l1(loc[2:3], g_w), 0.0)
          + jnp.where(pos, _smooth_l1(loc[3:4], g_h), 0.0))
    ll_s = jnp.sum(ll, axis=1, keepdims=True)

    lmd = landm_ref[0]
    llm = jnp.zeros((1, P), jnp.float32)
    for i in range(5):
        gx = (matched[4 + 2 * i:5 + 2 * i] - pcx) / (_V0 * pw)
        gy = (matched[5 + 2 * i:6 + 2 * i] - pcy) / (_V0 * ph)
        llm = llm + jnp.where(pos1, _smooth_l1(lmd[2 * i:2 * i + 1], gx), 0.0)
        llm = llm + jnp.where(pos1, _smooth_l1(lmd[2 * i + 1:2 * i + 2], gy), 0.0)
    llm_s = jnp.sum(llm, axis=1, keepdims=True)

    gd = gender_ref[0]
    g0, g1 = gd[0:1], gd[1:2]
    gm = jnp.maximum(g0, g1)
    e0 = jnp.exp(g0 - gm)
    e1 = jnp.exp(g1 - gm)
    es = e0 + e1
    p0 = jnp.clip(e0 / es, 1e-12, 1.0 - 1e-12)
    p1 = jnp.clip(e1 / es, 1e-12, 1.0 - 1e-12)
    w1 = gender_g
    w0 = 1.0 - gender_g
    bce = -(w0 * jnp.log(p0) + w1 * jnp.log(1.0 - p0)
            + w1 * jnp.log(p1) + w0 * jnp.log(1.0 - p1))
    lg_s = jnp.sum(jnp.where(pos1, bce, 0.0), axis=1, keepdims=True)

    cf = conf_ref[0]
    c0, c1 = cf[0:1], cf[1:2]
    cm = jnp.maximum(c0, c1)
    lse = cm + jnp.log(jnp.exp(c0 - cm) + jnp.exp(c1 - cm))
    cep_s = jnp.sum(jnp.where(pos, lse - c1, 0.0), axis=1, keepdims=True)
    mlc_ref[0] = jnp.where(pos, 0.0, lse - c0)

    npos_s = jnp.sum(pos.astype(jnp.float32), axis=1, keepdims=True)
    n1_s = jnp.sum(pos1.astype(jnp.float32), axis=1, keepdims=True)
    z = jnp.zeros((1, 1), jnp.float32)
    stats_ref[0] = jnp.concatenate(
        [ll_s, llm_s, lg_s, cep_s, npos_s, n1_s, z, z], axis=1)


def _mine_kernel(stats_ref, mlc_ref, out_ref):
    stats = stats_ref[...]
    v = mlc_ref[...]
    B = stats.shape[0]
    npos = stats[:, 4:5]
    k = jnp.minimum(npos * _NEGPOS_RATIO, float(v.shape[1] - 1))
    kint = k.astype(jnp.int32)
    bits = jax.lax.bitcast_convert_type(v, jnp.int32)

    def body(_, lohi):
        lo, hi = lohi
        mid = lo + (hi - lo + 1) // 2
        cnt = jnp.sum((bits >= mid).astype(jnp.int32), axis=1, keepdims=True)
        ge = cnt >= kint
        return jnp.where(ge, mid, lo), jnp.where(ge, hi, mid - 1)

    lo0 = jnp.zeros((B, 1), jnp.int32)
    hi0 = jnp.full((B, 1), 0x7F7FFFFF, jnp.int32)
    t, _ = jax.lax.fori_loop(0, 31, body, (lo0, hi0))

    gt = bits > t
    cnt_gt = jnp.sum(gt.astype(jnp.float32), axis=1, keepdims=True)
    sum_gt = jnp.sum(jnp.where(gt, v, 0.0), axis=1, keepdims=True)
    tval = jax.lax.bitcast_convert_type(t, jnp.float32)
    topk = sum_gt + (k - cnt_gt) * tval

    neg_total = jnp.sum(topk, axis=0, keepdims=True)
    ll = jnp.sum(stats[:, 0:1], axis=0, keepdims=True)
    llm = jnp.sum(stats[:, 1:2], axis=0, keepdims=True)
    lg = jnp.sum(stats[:, 2:3], axis=0, keepdims=True)
    cep = jnp.sum(stats[:, 3:4], axis=0, keepdims=True)
    n = jnp.maximum(jnp.sum(stats[:, 4:5], axis=0, keepdims=True), 1.0)
    n1 = jnp.maximum(jnp.sum(stats[:, 5:6], axis=0, keepdims=True), 1.0)
    out_ref[...] = jnp.concatenate(
        [ll / n, (cep + neg_total) / n, llm / n1, lg / n1], axis=1)


@functools.partial(jax.jit, static_argnames=())
def kernel(loc_data, conf_data, landm_data, gender_data, priors, targets):
    B, P, _ = loc_data.shape
    T = targets.shape[1]
    locT = loc_data.transpose(0, 2, 1)
    confT = conf_data.transpose(0, 2, 1)
    landmT = landm_data.transpose(0, 2, 1)
    genderT = gender_data.transpose(0, 2, 1)
    priorsT = priors.T
    targetsT = targets.transpose(0, 2, 1)

    stats, mlc = pl.pallas_call(
        _match_kernel,
        grid=(B,),
        in_specs=[
            pl.BlockSpec((1, 4, P), lambda b: (b, 0, 0)),
            pl.BlockSpec((1, _NUM_CLASSES, P), lambda b: (b, 0, 0)),
            pl.BlockSpec((1, 10, P), lambda b: (b, 0, 0)),
            pl.BlockSpec((1, 2, P), lambda b: (b, 0, 0)),
            pl.BlockSpec((4, P), lambda b: (0, 0)),
            pl.BlockSpec((1, T, 16), lambda b: (b, 0, 0)),
            pl.BlockSpec((1, 16, T), lambda b: (b, 0, 0)),
        ],
        out_specs=[
            pl.BlockSpec((1, 1, 8), lambda b: (b, 0, 0)),
            pl.BlockSpec((1, 1, P), lambda b: (b, 0, 0)),
        ],
        out_shape=[
            jax.ShapeDtypeStruct((B, 1, 8), jnp.float32),
            jax.ShapeDtypeStruct((B, 1, P), jnp.float32),
        ],
    )(locT, confT, landmT, genderT, priorsT, targets, targetsT)

    out = pl.pallas_call(
        _mine_kernel,
        in_specs=[
            pl.BlockSpec((B, 8), lambda: (0, 0)),
            pl.BlockSpec((B, P), lambda: (0, 0)),
        ],
        out_specs=pl.BlockSpec((1, 4), lambda: (0, 0)),
        out_shape=jax.ShapeDtypeStruct((1, 4), jnp.float32),
    )(stats.reshape(B, 8), mlc.reshape(B, P))

    return (out[0, 0], out[0, 1], out[0, 2], out[0, 3])


# speedup vs baseline: 122.5694x; 1.1714x over previous
import functools

import jax
import jax.numpy as jnp
from jax.experimental import pallas as pl

_NUM_CLASSES = 2
_THRESHOLD = 0.35
_NEGPOS_RATIO = 7.0
_V0 = 0.1
_V1 = 0.2


def _smooth_l1(x, t):
    d = x - t
    ad = jnp.abs(d)
    return jnp.where(ad < 1.0, 0.5 * d * d, ad - 0.5)


def _match_kernel(loc_ref, conf_ref, landm_ref, gender_ref, pri_ref, tgt_ref,
                  aug_ref, ca_ref, da_ref, cb_ref, db_ref, stats_ref, mlc_ref):
    T, P = tgt_ref.shape[1], pri_ref.shape[1]
    pri = pri_ref[...]
    tgtb = tgt_ref[0]
    aug = aug_ref[0]

    pcx, pcy = pri[0:1, :], pri[1:2, :]
    pw, ph = pri[2:3, :], pri[3:4, :]
    xb1 = pcx - pw / 2
    yb1 = pcy - ph / 2
    xb2 = pcx + pw / 2
    yb2 = pcy + ph / 2

    tx1, ty1 = tgtb[:, 0:1], tgtb[:, 1:2]
    tx2, ty2 = tgtb[:, 2:3], tgtb[:, 3:4]

    ix1 = jnp.maximum(tx1, xb1)
    iy1 = jnp.maximum(ty1, yb1)
    ix2 = jnp.minimum(tx2, xb2)
    iy2 = jnp.minimum(ty2, yb2)
    inter = jnp.maximum(ix2 - ix1, 0.0) * jnp.maximum(iy2 - iy1, 0.0)
    area_t = (tx2 - tx1) * (ty2 - ty1)
    area_p = (xb2 - xb1) * (yb2 - yb1)
    ov = inter / (area_t + area_p - inter)

    lane_i = jax.lax.broadcasted_iota(jnp.int32, (T, P), 1)
    sub_i = jax.lax.broadcasted_iota(jnp.int32, (T, P), 0)

    bpo = jnp.max(ov, axis=1, keepdims=True)
    bpi = jnp.min(jnp.where(ov == bpo, lane_i, P), axis=1, keepdims=True)
    valid = (bpo >= 0.2).astype(jnp.int32)

    bto = jnp.max(ov, axis=0, keepdims=True)
    bti = jnp.min(jnp.where(ov == bto, sub_i, T), axis=0, keepdims=True)

    eq = bpi == lane_i
    code = jnp.where(eq, 2 * sub_i + valid, -1)
    last_code = jnp.max(code, axis=0, keepdims=True)
    any_eq = last_code >= 0
    last_j = last_code // 2
    valid_last = (last_code - 2 * last_j) == 1
    bto2 = jnp.where(any_eq & valid_last, 2.0, bto)
    bti2 = jnp.where(any_eq, last_j, bti)

    onehot = (bti2 == sub_i).astype(jnp.float32)
    g24 = jax.lax.dot_general(
        aug, onehot, (((1,), (0,)), ((), ())),
        precision=jax.lax.Precision.HIGHEST,
        preferred_element_type=jnp.float32)

    label_g = g24[4:5]
    gender_g = g24[5:6]
    conf = jnp.where(bto2 < _THRESHOLD, 0.0, label_g)
    pos = conf != 0.0
    pos1 = conf > 0.0

    va = (g24[0:8] - da_ref[...]) / ca_ref[...]
    row8 = jax.lax.broadcasted_iota(jnp.int32, (8, P), 0)
    is_log = (row8 == 2) | (row8 == 3)
    ta = jnp.where(is_log, jnp.log(jnp.where(is_log, va, 1.0)) / _V1, va)
    loc = loc_ref[0]
    ll = jnp.where(pos, _smooth_l1(loc, ta[0:4]), 0.0)
    ll_s = jnp.sum(ll, axis=(0, 1), keepdims=True)

    tb = (g24[8:18] - db_ref[...]) / cb_ref[...]
    lmd = landm_ref[0]
    llm = jnp.where(pos1, _smooth_l1(lmd, tb), 0.0)
    llm_s = jnp.sum(llm, axis=(0, 1), keepdims=True)

    gd = gender_ref[0]
    g0, g1 = gd[0:1], gd[1:2]
    gm = jnp.maximum(g0, g1)
    e0 = jnp.exp(g0 - gm)
    e1 = jnp.exp(g1 - gm)
    es = e0 + e1
    p0 = jnp.clip(e0 / es, 1e-12, 1.0 - 1e-12)
    p1 = jnp.clip(e1 / es, 1e-12, 1.0 - 1e-12)
    w1 = gender_g
    w0 = 1.0 - gender_g
    bce = -(w0 * jnp.log(p0) + w1 * jnp.log(1.0 - p0)
            + w1 * jnp.log(p1) + w0 * jnp.log(1.0 - p1))
    lg_s = jnp.sum(jnp.where(pos1, bce, 0.0), axis=1, keepdims=True)

    cf = conf_ref[0]
    c0, c1 = cf[0:1], cf[1:2]
    cm = jnp.maximum(c0, c1)
    lse = cm + jnp.log(jnp.exp(c0 - cm) + jnp.exp(c1 - cm))
    cep_s = jnp.sum(jnp.where(pos, lse - c1, 0.0), axis=1, keepdims=True)
    mlc_ref[0] = jnp.where(pos, 0.0, lse - c0)

    npos_s = jnp.sum(pos.astype(jnp.float32), axis=1, keepdims=True)
    n1_s = jnp.sum(pos1.astype(jnp.float32), axis=1, keepdims=True)
    z = jnp.zeros((1, 1), jnp.float32)
    stats_ref[0] = jnp.concatenate(
        [ll_s, llm_s, lg_s, cep_s, npos_s, n1_s, z, z], axis=1)


def _mine_kernel(stats_ref, mlc_ref, out_ref):
    stats = stats_ref[...]
    v = mlc_ref[...]
    B = stats.shape[0]
    npos = stats[:, 4:5]
    k = jnp.minimum(npos * _NEGPOS_RATIO, float(v.shape[1] - 1))
    kint = k.astype(jnp.int32)
    bits = jax.lax.bitcast_convert_type(v, jnp.int32)

    def body(_, lohi):
        lo, hi = lohi
        mid = lo + (hi - lo + 1) // 2
        cnt = jnp.sum((bits >= mid).astype(jnp.int32), axis=1, keepdims=True)
        ge = cnt >= kint
        return jnp.where(ge, mid, lo), jnp.where(ge, hi, mid - 1)

    lo0 = jnp.zeros((B, 1), jnp.int32)
    hi0 = jnp.full((B, 1), 0x7F7FFFFF, jnp.int32)
    t, _ = jax.lax.fori_loop(0, 31, body, (lo0, hi0))

    gt = bits > t
    cnt_gt = jnp.sum(gt.astype(jnp.float32), axis=1, keepdims=True)
    sum_gt = jnp.sum(jnp.where(gt, v, 0.0), axis=1, keepdims=True)
    tval = jax.lax.bitcast_convert_type(t, jnp.float32)
    topk = sum_gt + (k - cnt_gt) * tval

    neg_total = jnp.sum(topk, axis=0, keepdims=True)
    ll = jnp.sum(stats[:, 0:1], axis=0, keepdims=True)
    llm = jnp.sum(stats[:, 1:2], axis=0, keepdims=True)
    lg = jnp.sum(stats[:, 2:3], axis=0, keepdims=True)
    cep = jnp.sum(stats[:, 3:4], axis=0, keepdims=True)
    n = jnp.maximum(jnp.sum(stats[:, 4:5], axis=0, keepdims=True), 1.0)
    n1 = jnp.maximum(jnp.sum(stats[:, 5:6], axis=0, keepdims=True), 1.0)
    out_ref[...] = jnp.concatenate(
        [ll / n, (cep + neg_total) / n, llm / n1, lg / n1], axis=1)


@functools.partial(jax.jit, static_argnames=())
def kernel(loc_data, conf_data, landm_data, gender_data, priors, targets):
    B, P, _ = loc_data.shape
    T = targets.shape[1]
    locT = loc_data.transpose(0, 2, 1)
    confT = conf_data.transpose(0, 2, 1)
    landmT = landm_data.transpose(0, 2, 1)
    genderT = gender_data.transpose(0, 2, 1)
    priorsT = priors.T

    tT = targets.transpose(0, 2, 1)
    zrow = jnp.zeros((B, 1, T), jnp.float32)
    aug = jnp.concatenate([
        (tT[:, 0:1] + tT[:, 2:3]) / 2,
        (tT[:, 1:2] + tT[:, 3:4]) / 2,
        tT[:, 2:3] - tT[:, 0:1],
        tT[:, 3:4] - tT[:, 1:2],
        tT[:, 14:15],
        tT[:, 15:16],
        zrow, zrow,
        tT[:, 4:14],
        jnp.zeros((B, 6, T), jnp.float32),
    ], axis=1)

    pw, ph = priorsT[2:3], priorsT[3:4]
    pcx, pcy = priorsT[0:1], priorsT[1:2]
    one = jnp.ones((1, P), jnp.float32)
    zero = jnp.zeros((1, P), jnp.float32)
    ca = jnp.concatenate([_V0 * pw, _V0 * ph, pw, ph, one, one, one, one], 0)
    da = jnp.concatenate([pcx, pcy] + [zero] * 6, 0)
    cb = jnp.concatenate([_V0 * pw, _V0 * ph] * 5, 0)
    db = jnp.concatenate([pcx, pcy] * 5, 0)

    stats, mlc = pl.pallas_call(
        _match_kernel,
        grid=(B,),
        in_specs=[
            pl.BlockSpec((1, 4, P), lambda b: (b, 0, 0)),
            pl.BlockSpec((1, _NUM_CLASSES, P), lambda b: (b, 0, 0)),
            pl.BlockSpec((1, 10, P), lambda b: (b, 0, 0)),
            pl.BlockSpec((1, 2, P), lambda b: (b, 0, 0)),
            pl.BlockSpec((4, P), lambda b: (0, 0)),
            pl.BlockSpec((1, T, 16), lambda b: (b, 0, 0)),
            pl.BlockSpec((1, 24, T), lambda b: (b, 0, 0)),
            pl.BlockSpec((8, P), lambda b: (0, 0)),
            pl.BlockSpec((8, P), lambda b: (0, 0)),
            pl.BlockSpec((10, P), lambda b: (0, 0)),
            pl.BlockSpec((10, P), lambda b: (0, 0)),
        ],
        out_specs=[
            pl.BlockSpec((1, 1, 8), lambda b: (b, 0, 0)),
            pl.BlockSpec((1, 1, P), lambda b: (b, 0, 0)),
        ],
        out_shape=[
            jax.ShapeDtypeStruct((B, 1, 8), jnp.float32),
            jax.ShapeDtypeStruct((B, 1, P), jnp.float32),
        ],
    )(locT, confT, landmT, genderT, priorsT, targets, aug, ca, da, cb, db)

    out = pl.pallas_call(
        _mine_kernel,
        in_specs=[
            pl.BlockSpec((B, 8), lambda: (0, 0)),
            pl.BlockSpec((B, P), lambda: (0, 0)),
        ],
        out_specs=pl.BlockSpec((1, 4), lambda: (0, 0)),
        out_shape=jax.ShapeDtypeStruct((1, 4), jnp.float32),
    )(stats.reshape(B, 8), mlc.reshape(B, P))

    return (out[0, 0], out[0, 1], out[0, 2], out[0, 3])
